# baseline (speedup 1.0000x reference)
import functools

import jax
import jax.numpy as jnp
from jax import lax
from jax.experimental import pallas as pl
from jax.experimental.pallas import tpu as pltpu
from jax.experimental.pallas import tpu_sc as plsc

EMBED_DIM = 64
BATCH = 16384
ROW_W = 128
IDX_CHUNK = 128
PASS_ROWS = 256


@functools.cache
def _build(num_users: int, num_items: int):
    info = plsc.get_sparse_core_info()
    nc, ns, lanes = info.num_cores, info.num_subcores, info.num_lanes
    nw = nc * ns
    b_per_w = BATCH // nw
    n_pass = b_per_w // PASS_ROWS
    n_chunks = PASS_ROWS // IDX_CHUNK
    n_groups = PASS_ROWS // lanes

    mesh = plsc.VectorSubcoreMesh(core_axis_name="c", subcore_axis_name="s")

    @functools.partial(
        pl.kernel,
        out_type=jax.ShapeDtypeStruct((BATCH,), jnp.float32),
        mesh=mesh,
        compiler_params=pltpu.CompilerParams(needs_layout_passes=False),
        scratch_types=[
            pltpu.VMEM((n_chunks, IDX_CHUNK), jnp.int32),
            pltpu.VMEM((n_chunks, IDX_CHUNK), jnp.int32),
            pltpu.VMEM((b_per_w,), jnp.int32),
            pltpu.VMEM((b_per_w,), jnp.int32),
            pltpu.VMEM((PASS_ROWS, ROW_W), jnp.float32),
            pltpu.VMEM((PASS_ROWS, ROW_W), jnp.float32),
            pltpu.VMEM((b_per_w,), jnp.float32),
            pltpu.SemaphoreType.DMA,
        ],
    )
    def sc_kernel(urow_hbm, irow_hbm, uoff_hbm, ioff_hbm,
                  utab_hbm, itab_hbm, out_hbm,
                  uidx, iidx, uoffv, ioffv, urows, irows, outv, sem):
        wid = lax.axis_index("s") * nc + lax.axis_index("c")
        base = wid * b_per_w

        pltpu.sync_copy(uoff_hbm.at[pl.ds(base, b_per_w)], uoffv)
        pltpu.sync_copy(ioff_hbm.at[pl.ds(base, b_per_w)], ioffv)

        lane = jax.lax.iota(jnp.int32, lanes)

        for p in range(n_pass):
            pbase = base + p * PASS_ROWS
            for j in range(n_chunks):
                pltpu.sync_copy(
                    urow_hbm.at[pl.ds(pbase + j * IDX_CHUNK, IDX_CHUNK)],
                    uidx.at[j])
                pltpu.sync_copy(
                    irow_hbm.at[pl.ds(pbase + j * IDX_CHUNK, IDX_CHUNK)],
                    iidx.at[j])

            copies = []
            for j in range(n_chunks):
                dst = pl.ds(j * IDX_CHUNK, IDX_CHUNK)
                copies.append(pltpu.async_copy(utab_hbm.at[uidx.at[j]],
                                               urows.at[dst], sem))
                copies.append(pltpu.async_copy(itab_hbm.at[iidx.at[j]],
                                               irows.at[dst], sem))
            for c in copies:
                c.wait()

            def group_body(g, _, p=p):
                row = g * lanes + lane
                ucol0 = uoffv[pl.ds(p * PASS_ROWS + g * lanes, lanes)]
                icol0 = ioffv[pl.ds(p * PASS_ROWS + g * lanes, lanes)]
                acc = jnp.zeros((lanes,), jnp.float32)
                for d in range(EMBED_DIM):
                    u = plsc.load_gather(urows, [row, ucol0 + d])
                    v = plsc.load_gather(irows, [row, icol0 + d])
                    acc = acc + u * v
                outv[pl.ds(p * PASS_ROWS + g * lanes, lanes)] = acc
                return 0

            lax.fori_loop(0, n_groups, group_body, 0)

        pltpu.sync_copy(outv, out_hbm.at[pl.ds(base, b_per_w)])

    return sc_kernel


def kernel(user, item, user_table, item_table):
    nu, nd = user_table.shape
    ni, _ = item_table.shape
    fn = _build(nu, ni)
    user = user.astype(jnp.int32)
    item = item.astype(jnp.int32)
    pack = ROW_W // nd
    return fn(
        user // pack, item // pack,
        (user % pack) * nd, (item % pack) * nd,
        user_table.reshape(nu // pack, ROW_W),
        item_table.reshape(ni // pack, ROW_W),
    )
